# baseline (speedup 1.0000x reference)
import functools

import jax
import jax.numpy as jnp
from jax import lax
from jax.experimental import pallas as pl
from jax.experimental.pallas import tpu as pltpu
from jax.experimental.pallas import tpu_sc as plsc

BATCH = 4096
SEQ = 50
EMB = 32
LANES = 16
NC, NS = 2, 16
NW = NC * NS
BPW = BATCH // NW
RPC = 2
CHUNKS = BPW // RPC
IPC = RPC * SEQ
IPAD = 104
NBUF = 4
INV_SEQ = 1.0 / SEQ


def _body(idx_hbm, table_hbm, out_hbm, idx_v, rows_v, out_v, sems):
    wid = lax.axis_index("s") * NC + lax.axis_index("c")

    pltpu.sync_copy(idx_hbm.at[wid], idx_v)

    def gather(c, slot):
        pltpu.async_copy(table_hbm.at[idx_v.at[c]], rows_v.at[slot], sems.at[slot])

    for b in range(NBUF):
        gather(b, b)

    def accumulate(slot, c):
        for r in range(RPC):
            base = r * SEQ
            acc0 = rows_v[slot, base, pl.ds(0, LANES)]
            acc1 = rows_v[slot, base, pl.ds(LANES, LANES)]
            for s in range(1, SEQ):
                acc0 = acc0 + rows_v[slot, base + s, pl.ds(0, LANES)]
                acc1 = acc1 + rows_v[slot, base + s, pl.ds(LANES, LANES)]
            out_row = c * RPC + r
            out_v[out_row, pl.ds(0, LANES)] = acc0 * INV_SEQ
            out_v[out_row, pl.ds(LANES, LANES)] = acc1 * INV_SEQ

    def step(i, carry):
        for b in range(NBUF):
            c = i * NBUF + b
            pltpu.make_async_copy(
                table_hbm.at[idx_v.at[c]], rows_v.at[b], sems.at[b]
            ).wait()
            nxt = c + NBUF

            @pl.when(nxt < CHUNKS)
            def _():
                gather(nxt, b)

            accumulate(b, c)
        return carry

    lax.fori_loop(0, CHUNKS // NBUF, step, 0)

    pltpu.sync_copy(out_v, out_hbm.at[pl.ds(wid * BPW, BPW)])


def kernel(inputs, table):
    idx = inputs.astype(jnp.int32).reshape(NW, CHUNKS, IPC)
    idx = jnp.pad(idx, ((0, 0), (0, 0), (0, IPAD - IPC)))

    mesh = plsc.VectorSubcoreMesh(core_axis_name="c", subcore_axis_name="s")
    run = pl.kernel(
        _body,
        out_type=jax.ShapeDtypeStruct((BATCH, EMB), jnp.float32),
        mesh=mesh,
        scratch_types=[
            pltpu.VMEM((CHUNKS, IPAD), jnp.int32),
            pltpu.VMEM((NBUF, IPAD, EMB), jnp.float32),
            pltpu.VMEM((BPW, EMB), jnp.float32),
            pltpu.SemaphoreType.DMA((NBUF,)),
        ],
        compiler_params=pltpu.CompilerParams(use_tc_tiling_on_sc=False),
    )
    return run(idx, table)
